# baseline (speedup 1.0000x reference)
import functools

import jax
import jax.numpy as jnp
from jax import lax
from jax.experimental import pallas as pl
from jax.experimental.pallas import tpu as pltpu
from jax.experimental.pallas import tpu_sc as plsc

VOCAB = 1000000
DIM = 64
B = 4096 * 200
NC = 2
NS = 16
NW = NC * NS
BPW = B // NW
CH = 128
NCHUNK = BPW // CH


def _emb_body(x_hbm, table_hbm, out_hbm, idx_v, rows_v, gsem):
    wid = lax.axis_index("s") * NC + lax.axis_index("c")
    base = wid * BPW

    pltpu.sync_copy(x_hbm.at[wid], idx_v)

    def step(j, carry):
        pltpu.async_copy(table_hbm.at[idx_v.at[j]], rows_v, gsem).wait()
        pltpu.sync_copy(rows_v, out_hbm.at[pl.ds(base + j * CH, CH)])
        return carry

    lax.fori_loop(0, NCHUNK, step, 0)


@jax.jit
def _emb(x3, table):
    mesh = plsc.VectorSubcoreMesh(core_axis_name="c", subcore_axis_name="s")
    kern = functools.partial(
        pl.kernel,
        out_type=jax.ShapeDtypeStruct((B, DIM), jnp.float32),
        mesh=mesh,
        scratch_types=[
            pltpu.VMEM((NCHUNK, CH), jnp.int32),
            pltpu.VMEM((CH, DIM), jnp.float32),
            pltpu.SemaphoreType.DMA,
        ],
        compiler_params=pltpu.CompilerParams(use_tc_tiling_on_sc=False),
    )(_emb_body)
    return kern(x3, table)


def kernel(x, table):
    x3 = x.reshape(NW, NCHUNK, CH).astype(jnp.int32)
    out = _emb(x3, table)
    return out.reshape(4096, 200, DIM)


# speedup vs baseline: 1.1151x; 1.1151x over previous
import functools

import jax
import jax.numpy as jnp
from jax import lax
from jax.experimental import pallas as pl
from jax.experimental.pallas import tpu as pltpu
from jax.experimental.pallas import tpu_sc as plsc

VOCAB = 1000000
DIM = 64
B = 4096 * 200
NC = 2
NS = 16
NW = NC * NS
BPW = B // NW
CH = 128
NCHUNK = BPW // CH
K = 4
T = NCHUNK // K


def _emb_body(x_hbm, table_hbm, out_hbm, idx_v, rows_v, gs0, gs1, os0, os1):
    wid = lax.axis_index("s") * NC + lax.axis_index("c")
    base = wid * BPW

    pltpu.sync_copy(x_hbm.at[wid], idx_v)

    def fire_gathers(t, g, sem):
        for b in range(K):
            pltpu.async_copy(
                table_hbm.at[idx_v.at[t * K + b]], rows_v.at[g, b], sem)

    def drain_gathers(t, g, sem):
        for b in range(K):
            pltpu.make_async_copy(
                table_hbm.at[idx_v.at[t * K + b]], rows_v.at[g, b], sem).wait()

    def fire_outs(t, g, sem):
        for b in range(K):
            pltpu.async_copy(
                rows_v.at[g, b],
                out_hbm.at[pl.ds(base + (t * K + b) * CH, CH)], sem)

    def drain_outs(t, g, sem):
        for b in range(K):
            pltpu.make_async_copy(
                rows_v.at[g, b],
                out_hbm.at[pl.ds(base + (t * K + b) * CH, CH)], sem).wait()

    def halfstep(t, g, gsem_own, gsem_other, osem_own, osem_other, first):
        if first:
            @pl.when(t > 0)
            def _():
                drain_outs(t - 1, 1 - g, osem_other)
        else:
            drain_outs(t - 1, 1 - g, osem_other)

        @pl.when(t + 1 < T)
        def _():
            fire_gathers(t + 1, 1 - g, gsem_other)

        drain_gathers(t, g, gsem_own)
        fire_outs(t, g, osem_own)

    fire_gathers(0, 0, gs0)

    def body(i, carry):
        halfstep(2 * i, 0, gs0, gs1, os0, os1, True)
        halfstep(2 * i + 1, 1, gs1, gs0, os1, os0, False)
        return carry

    lax.fori_loop(0, T // 2, body, 0)
    drain_outs(T - 1, 1, os1)


@jax.jit
def _emb(x3, table):
    mesh = plsc.VectorSubcoreMesh(core_axis_name="c", subcore_axis_name="s")
    kern = functools.partial(
        pl.kernel,
        out_type=jax.ShapeDtypeStruct((B, DIM), jnp.float32),
        mesh=mesh,
        scratch_types=[
            pltpu.VMEM((NCHUNK, CH), jnp.int32),
            pltpu.VMEM((2, K, CH, DIM), jnp.float32),
            pltpu.SemaphoreType.DMA,
            pltpu.SemaphoreType.DMA,
            pltpu.SemaphoreType.DMA,
            pltpu.SemaphoreType.DMA,
        ],
        compiler_params=pltpu.CompilerParams(use_tc_tiling_on_sc=False),
    )(_emb_body)
    return kern(x3, table)


def kernel(x, table):
    x3 = x.reshape(NW, NCHUNK, CH).astype(jnp.int32)
    out = _emb(x3, table)
    return out.reshape(4096, 200, DIM)
